# baseline (speedup 1.0000x reference)
import functools

import jax
import jax.numpy as jnp
from jax import lax
from jax.experimental import pallas as pl
from jax.experimental.pallas import tpu as pltpu
from jax.experimental.pallas import tpu_sc as plsc

N = 10000
N2 = N // 2
E = 320000
D = 128
DH = D // 2

NUM_TILES = 16
CHUNK = 64
NUM_CHUNKS = 320
EDGES_PER_TILE = CHUNK * NUM_CHUNKS
E_PAD = EDGES_PER_TILE * NUM_TILES
NBS = 16
NUM_ROUNDS = NUM_CHUNKS // NBS
NPAIR = NBS // 2
ROWS2_PER_TILE = 312
TAIL2_ROW0 = ROWS2_PER_TILE * NUM_TILES
TAIL2_ROWS = N2 - TAIL2_ROW0

ROW_BLOCK = 2000


def _mm_body(x_self_ref, x_other_ref, wself_ref, wrel_ref, bias_ref,
             base_ref, table_ref):
    xs = x_self_ref[0]
    xo = x_other_ref[0]
    base_ref[0] = (
        jnp.dot(xs, wself_ref[0], preferred_element_type=jnp.float32)
        + bias_ref[0]
    )
    table_ref[0] = jnp.dot(xo, wrel_ref[0], preferred_element_type=jnp.float32)


def _tc_matmuls(x_cat, wself, wrel, bias):
    grid = (2, N // ROW_BLOCK)
    return pl.pallas_call(
        _mm_body,
        grid=grid,
        in_specs=[
            pl.BlockSpec((1, ROW_BLOCK, D), lambda c, r: (c, r, 0)),
            pl.BlockSpec((1, ROW_BLOCK, D), lambda c, r: (1 - c, r, 0)),
            pl.BlockSpec((1, D, D), lambda c, r: (c, 0, 0)),
            pl.BlockSpec((1, D, D), lambda c, r: (c, 0, 0)),
            pl.BlockSpec((1, 1, D), lambda c, r: (c, 0, 0)),
        ],
        out_specs=[
            pl.BlockSpec((1, ROW_BLOCK, D), lambda c, r: (c, r, 0)),
            pl.BlockSpec((1, ROW_BLOCK, D), lambda c, r: (c, r, 0)),
        ],
        out_shape=[
            jax.ShapeDtypeStruct((2, N, D), jnp.float32),
            jax.ShapeDtypeStruct((2, N, D), jnp.float32),
        ],
    )(x_cat, x_cat, wself, wrel, bias)


def _sc_body(table_hbm, base_hbm, src2_hbm, dst2_hbm, par_hbm, val_hbm,
             out_hbm, src2_v, dst2_v, par_v, val_v, buf_a, buf_b,
             table_sh, acc_sh, g_a, g_b):
    c = lax.axis_index("c")
    s = lax.axis_index("s")
    row0 = s * ROWS2_PER_TILE
    zero16 = jnp.zeros((16,), jnp.float32)

    def scale_pack(j, buf):
        for grp in range(CHUNK // 16):
            vv = val_v[j, pl.ds(grp * 16, 16)]
            pv = par_v[j, pl.ds(grp * 16, 16)]
            for i in range(16):
                e = grp * 16 + i
                v = vv[i]
                p = pv[i]
                off_s = (p & 2) * 32
                off_d = (p & 1) * 64
                off_z = 64 - off_d
                regs = [buf[e, pl.ds(off_s + 16 * k, 16)]
                        for k in range(DH // 16)]
                for k in range(DH // 16):
                    buf[e, pl.ds(off_d + 16 * k, 16)] = regs[k] * v
                for k in range(DH // 16):
                    buf[e, pl.ds(off_z + 16 * k, 16)] = zero16

    def half_body(h):
        pltpu.sync_copy(base_hbm.at[c, h, pl.ds(row0, ROWS2_PER_TILE)],
                        acc_sh.at[pl.ds(row0, ROWS2_PER_TILE)])
        pltpu.sync_copy(table_hbm.at[c, h, pl.ds(row0, ROWS2_PER_TILE)],
                        table_sh.at[pl.ds(row0, ROWS2_PER_TILE)])

        @pl.when(s == NUM_TILES - 1)
        def _init_tail():
            pltpu.sync_copy(base_hbm.at[c, h, pl.ds(TAIL2_ROW0, TAIL2_ROWS)],
                            acc_sh.at[pl.ds(TAIL2_ROW0, TAIL2_ROWS)])
            pltpu.sync_copy(table_hbm.at[c, h, pl.ds(TAIL2_ROW0, TAIL2_ROWS)],
                            table_sh.at[pl.ds(TAIL2_ROW0, TAIL2_ROWS)])

        plsc.subcore_barrier()

        def round_body(r, rcarry):
            r0 = pl.multiple_of(r * NBS, NBS)
            pltpu.sync_copy(src2_hbm.at[c, s, pl.ds(r0, NBS)], src2_v)
            pltpu.sync_copy(dst2_hbm.at[c, s, pl.ds(r0, NBS)], dst2_v)
            pltpu.sync_copy(par_hbm.at[c, s, pl.ds(r0, NBS)], par_v)
            pltpu.sync_copy(val_hbm.at[c, s, pl.ds(r0, NBS)], val_v)
            pltpu.async_copy(table_sh.at[src2_v.at[0]], buf_a, g_a)

            def pair_body(q, carry):
                k0 = 2 * q
                pltpu.make_async_copy(table_sh.at[src2_v.at[k0]], buf_a,
                                      g_a).wait()
                pltpu.async_copy(table_sh.at[src2_v.at[k0 + 1]], buf_b, g_b)
                scale_pack(k0, buf_a)
                pltpu.sync_copy(buf_a, acc_sh.at[dst2_v.at[k0]], add=True)
                pltpu.make_async_copy(table_sh.at[src2_v.at[k0 + 1]], buf_b,
                                      g_b).wait()

                @pl.when(k0 + 2 < NBS)
                def _prefetch_next():
                    pltpu.async_copy(table_sh.at[src2_v.at[k0 + 2]], buf_a,
                                     g_a)

                scale_pack(k0 + 1, buf_b)
                pltpu.sync_copy(buf_b, acc_sh.at[dst2_v.at[k0 + 1]],
                                add=True)
                return carry

            lax.fori_loop(0, NPAIR, pair_body, 0)
            return rcarry

        lax.fori_loop(0, NUM_ROUNDS, round_body, 0)

        plsc.subcore_barrier()
        pltpu.sync_copy(acc_sh.at[pl.ds(row0, ROWS2_PER_TILE)],
                        out_hbm.at[c, h, pl.ds(row0, ROWS2_PER_TILE)])

        @pl.when(s == NUM_TILES - 1)
        def _write_tail():
            pltpu.sync_copy(acc_sh.at[pl.ds(TAIL2_ROW0, TAIL2_ROWS)],
                            out_hbm.at[c, h, pl.ds(TAIL2_ROW0, TAIL2_ROWS)])

        plsc.subcore_barrier()

    half_body(0)
    half_body(1)


_sc_spmm = functools.partial(
    pl.kernel,
    out_type=jax.ShapeDtypeStruct((2, 2, N2, D), jnp.float32),
    mesh=plsc.VectorSubcoreMesh(core_axis_name="c", subcore_axis_name="s"),
    scratch_types=[
        pltpu.VMEM((NBS, CHUNK), jnp.int32),
        pltpu.VMEM((NBS, CHUNK), jnp.int32),
        pltpu.VMEM((NBS, CHUNK), jnp.int32),
        pltpu.VMEM((NBS, CHUNK), jnp.float32),
        pltpu.VMEM((CHUNK, D), jnp.float32),
        pltpu.VMEM((CHUNK, D), jnp.float32),
        pltpu.VMEM_SHARED((N2, D), jnp.float32),
        pltpu.VMEM_SHARED((N2, D), jnp.float32),
        pltpu.SemaphoreType.DMA,
        pltpu.SemaphoreType.DMA,
    ],
)(_sc_body)


def kernel(x_a, x_b, adj_ab_indices, adj_ab_values, adj_ba_indices,
           adj_ba_values, W_rel_ab, w_self_a, bias_a, W_rel_ba, w_self_b,
           bias_b):
    x_cat = jnp.stack([x_a, x_b])
    wself = jnp.stack([w_self_a, w_self_b]) * 0.5
    wrel = jnp.stack([W_rel_ab, W_rel_ba]) * 0.5
    bias = jnp.stack([bias_a, bias_b])

    base, table = _tc_matmuls(x_cat, wself, wrel, bias)

    def interleave(x):
        x5 = x.reshape(2, N2, 2, 2, DH)
        return x5.transpose(0, 3, 1, 2, 4).reshape(2, 2, N2, D)

    table_il = interleave(table)
    base_il = interleave(base)

    pad = E_PAD - E
    zpad_i = jnp.zeros((pad,), jnp.int32)
    zpad_f = jnp.zeros((pad,), jnp.float32)

    src = jnp.concatenate([adj_ab_indices[1].astype(jnp.int32), zpad_i,
                           adj_ba_indices[1].astype(jnp.int32), zpad_i])
    dst = jnp.concatenate([adj_ab_indices[0].astype(jnp.int32), zpad_i,
                           adj_ba_indices[0].astype(jnp.int32), zpad_i])
    val = jnp.concatenate([adj_ab_values, zpad_f, adj_ba_values, zpad_f])
    shape4 = (2, NUM_TILES, NUM_CHUNKS, CHUNK)
    src2 = (src >> 1).reshape(shape4)
    dst2 = (dst >> 1).reshape(shape4)
    par = ((src & 1) * 2 + (dst & 1)).reshape(shape4)
    valr = val.reshape(shape4)

    out = _sc_spmm(table_il, base_il, src2, dst2, par, valr)
    out = out.reshape(2, 2, N2, 2, DH).transpose(0, 2, 3, 1, 4)
    out = out.reshape(2, N, D)
    return (out[0], out[1])


# speedup vs baseline: 1.7381x; 1.7381x over previous
import functools

import jax
import jax.numpy as jnp
from jax import lax
from jax.experimental import pallas as pl
from jax.experimental.pallas import tpu as pltpu
from jax.experimental.pallas import tpu_sc as plsc

N = 10000
E = 320000
D = 128

NUM_TILES = 16
CHUNK = 128
NUM_CHUNKS = 160
EDGES_PER_TILE = CHUNK * NUM_CHUNKS
E_PAD = EDGES_PER_TILE * NUM_TILES
NB = 16
NUM_BLOCKS = NUM_CHUNKS // NB
ROWS_PER_TILE = 624
TAIL_ROW0 = ROWS_PER_TILE * NUM_TILES
TAIL_ROWS = N - TAIL_ROW0

ROW_BLOCK = 2000


def _mm_body(x_self_ref, x_other_ref, wself_ref, wrel_ref, bias_ref,
             base_ref, table_ref):
    xs = x_self_ref[0]
    xo = x_other_ref[0]
    base_ref[0] = (
        jnp.dot(xs, wself_ref[0], preferred_element_type=jnp.float32)
        + bias_ref[0]
    )
    table_ref[0] = jnp.dot(xo, wrel_ref[0], preferred_element_type=jnp.float32)


def _tc_matmuls(x_cat, wself, wrel, bias):
    grid = (2, N // ROW_BLOCK)
    return pl.pallas_call(
        _mm_body,
        grid=grid,
        in_specs=[
            pl.BlockSpec((1, ROW_BLOCK, D), lambda c, r: (c, r, 0)),
            pl.BlockSpec((1, ROW_BLOCK, D), lambda c, r: (1 - c, r, 0)),
            pl.BlockSpec((1, D, D), lambda c, r: (c, 0, 0)),
            pl.BlockSpec((1, D, D), lambda c, r: (c, 0, 0)),
            pl.BlockSpec((1, 1, D), lambda c, r: (c, 0, 0)),
        ],
        out_specs=[
            pl.BlockSpec((1, ROW_BLOCK, D), lambda c, r: (c, r, 0)),
            pl.BlockSpec((1, ROW_BLOCK, D), lambda c, r: (c, r, 0)),
        ],
        out_shape=[
            jax.ShapeDtypeStruct((2, N, D), jnp.float32),
            jax.ShapeDtypeStruct((2, N, D), jnp.float32),
        ],
    )(x_cat, x_cat, wself, wrel, bias)


def _sc_body(table_hbm, base_hbm, src_hbm, dst_hbm, val_hbm, out_hbm,
             src_v, dst_v, val_v, rows_a, rows_b, acc_sh, sem_a, sem_b):
    c = lax.axis_index("c")
    s = lax.axis_index("s")
    row0 = s * ROWS_PER_TILE

    pltpu.sync_copy(base_hbm.at[c, pl.ds(row0, ROWS_PER_TILE)],
                    acc_sh.at[pl.ds(row0, ROWS_PER_TILE)])

    @pl.when(s == NUM_TILES - 1)
    def _init_tail():
        pltpu.sync_copy(base_hbm.at[c, pl.ds(TAIL_ROW0, TAIL_ROWS)],
                        acc_sh.at[pl.ds(TAIL_ROW0, TAIL_ROWS)])
    plsc.subcore_barrier()

    def scale_chunk(k, rows_ref):
        def group_body(g, carry):
            vv = val_v[k, pl.ds(g * 16, 16)]
            for i in range(16):
                e = g * 16 + i
                v = vv[i]
                for q in range(D // 16):
                    sl = pl.ds(q * 16, 16)
                    rows_ref[e, sl] = rows_ref[e, sl] * v
            return carry

        lax.fori_loop(0, CHUNK // 16, group_body, 0)

    def block_body(b, carry):
        pltpu.sync_copy(src_hbm.at[c, s, pl.ds(b * NB, NB)], src_v)
        pltpu.sync_copy(dst_hbm.at[c, s, pl.ds(b * NB, NB)], dst_v)
        pltpu.sync_copy(val_hbm.at[c, s, pl.ds(b * NB, NB)], val_v)
        pltpu.async_copy(table_hbm.at[src_v.at[0]], rows_a, sem_a)

        def pair_body(q, carry2):
            k0 = 2 * q
            pltpu.make_async_copy(table_hbm.at[src_v.at[k0]], rows_a,
                                  sem_a).wait()
            pltpu.async_copy(table_hbm.at[src_v.at[k0 + 1]], rows_b, sem_b)
            scale_chunk(k0, rows_a)
            pltpu.sync_copy(rows_a, acc_sh.at[dst_v.at[k0]], add=True)
            pltpu.make_async_copy(table_hbm.at[src_v.at[k0 + 1]], rows_b,
                                  sem_b).wait()

            @pl.when(k0 + 2 < NB)
            def _prefetch_next():
                pltpu.async_copy(table_hbm.at[src_v.at[k0 + 2]], rows_a,
                                 sem_a)

            scale_chunk(k0 + 1, rows_b)
            pltpu.sync_copy(rows_b, acc_sh.at[dst_v.at[k0 + 1]], add=True)
            return carry2

        lax.fori_loop(0, NB // 2, pair_body, 0)
        return carry

    lax.fori_loop(0, NUM_BLOCKS, block_body, 0)
    plsc.subcore_barrier()
    pltpu.sync_copy(acc_sh.at[pl.ds(row0, ROWS_PER_TILE)],
                    out_hbm.at[c, pl.ds(row0, ROWS_PER_TILE)])

    @pl.when(s == NUM_TILES - 1)
    def _write_tail():
        pltpu.sync_copy(acc_sh.at[pl.ds(TAIL_ROW0, TAIL_ROWS)],
                        out_hbm.at[c, pl.ds(TAIL_ROW0, TAIL_ROWS)])


_sc_spmm = functools.partial(
    pl.kernel,
    out_type=jax.ShapeDtypeStruct((2, N, D), jnp.float32),
    mesh=plsc.VectorSubcoreMesh(core_axis_name="c", subcore_axis_name="s"),
    scratch_types=[
        pltpu.VMEM((NB, CHUNK), jnp.int32),
        pltpu.VMEM((NB, CHUNK), jnp.int32),
        pltpu.VMEM((NB, CHUNK), jnp.float32),
        pltpu.VMEM((CHUNK, D), jnp.float32),
        pltpu.VMEM((CHUNK, D), jnp.float32),
        pltpu.VMEM_SHARED((N, D), jnp.float32),
        pltpu.SemaphoreType.DMA,
        pltpu.SemaphoreType.DMA,
    ],
)(_sc_body)


def kernel(x_a, x_b, adj_ab_indices, adj_ab_values, adj_ba_indices,
           adj_ba_values, W_rel_ab, w_self_a, bias_a, W_rel_ba, w_self_b,
           bias_b):
    x_cat = jnp.stack([x_a, x_b])
    wself = jnp.stack([w_self_a, w_self_b]) * 0.5
    wrel = jnp.stack([W_rel_ab, W_rel_ba]) * 0.5
    bias = jnp.stack([bias_a, bias_b])

    base, table = _tc_matmuls(x_cat, wself, wrel, bias)
    table_flat = table.reshape(2 * N, D)

    pad = E_PAD - E
    zpad_i = jnp.zeros((pad,), jnp.int32)
    zpad_f = jnp.zeros((pad,), jnp.float32)
    src = jnp.stack([
        jnp.concatenate([adj_ab_indices[1].astype(jnp.int32), zpad_i]),
        jnp.concatenate([adj_ba_indices[1].astype(jnp.int32) + N, zpad_i]),
    ]).reshape(2, NUM_TILES, NUM_CHUNKS, CHUNK)
    dst = jnp.stack([
        jnp.concatenate([adj_ab_indices[0].astype(jnp.int32), zpad_i]),
        jnp.concatenate([adj_ba_indices[0].astype(jnp.int32), zpad_i]),
    ]).reshape(2, NUM_TILES, NUM_CHUNKS, CHUNK)
    val = jnp.stack([
        jnp.concatenate([adj_ab_values, zpad_f]),
        jnp.concatenate([adj_ba_values, zpad_f]),
    ]).reshape(2, NUM_TILES, NUM_CHUNKS, CHUNK)

    out = _sc_spmm(table_flat, base, src, dst, val)
    return (out[0], out[1])


# speedup vs baseline: 1.7536x; 1.0089x over previous
import functools

import jax
import jax.numpy as jnp
from jax import lax
from jax.experimental import pallas as pl
from jax.experimental.pallas import tpu as pltpu
from jax.experimental.pallas import tpu_sc as plsc

N = 10000
E = 320000
D = 128

NUM_TILES = 16
CHUNK = 128
NUM_CHUNKS = 160
EDGES_PER_TILE = CHUNK * NUM_CHUNKS
E_PAD = EDGES_PER_TILE * NUM_TILES
NB = 32
NUM_BLOCKS = NUM_CHUNKS // NB
ROWS_PER_TILE = 624
TAIL_ROW0 = ROWS_PER_TILE * NUM_TILES
TAIL_ROWS = N - TAIL_ROW0

ROW_BLOCK = 2000


def _mm_body(x_self_ref, x_other_ref, wself_ref, wrel_ref, bias_ref,
             base_ref, table_ref):
    xs = x_self_ref[0]
    xo = x_other_ref[0]
    base_ref[0] = (
        jnp.dot(xs, wself_ref[0], preferred_element_type=jnp.float32)
        + bias_ref[0]
    )
    table_ref[0] = jnp.dot(xo, wrel_ref[0], preferred_element_type=jnp.float32)


def _tc_matmuls(x_cat, wself, wrel, bias):
    grid = (2, N // ROW_BLOCK)
    return pl.pallas_call(
        _mm_body,
        grid=grid,
        in_specs=[
            pl.BlockSpec((1, ROW_BLOCK, D), lambda c, r: (c, r, 0)),
            pl.BlockSpec((1, ROW_BLOCK, D), lambda c, r: (1 - c, r, 0)),
            pl.BlockSpec((1, D, D), lambda c, r: (c, 0, 0)),
            pl.BlockSpec((1, D, D), lambda c, r: (c, 0, 0)),
            pl.BlockSpec((1, 1, D), lambda c, r: (c, 0, 0)),
        ],
        out_specs=[
            pl.BlockSpec((1, ROW_BLOCK, D), lambda c, r: (c, r, 0)),
            pl.BlockSpec((1, ROW_BLOCK, D), lambda c, r: (c, r, 0)),
        ],
        out_shape=[
            jax.ShapeDtypeStruct((2, N, D), jnp.float32),
            jax.ShapeDtypeStruct((2, N, D), jnp.float32),
        ],
    )(x_cat, x_cat, wself, wrel, bias)


def _sc_body(table_hbm, base_hbm, src_hbm, dst_hbm, val_hbm, out_hbm,
             src_v, dst_v, val_v, rows_a, rows_b, acc_sh, sem_a, sem_b):
    c = lax.axis_index("c")
    s = lax.axis_index("s")
    row0 = s * ROWS_PER_TILE

    pltpu.sync_copy(base_hbm.at[c, pl.ds(row0, ROWS_PER_TILE)],
                    acc_sh.at[pl.ds(row0, ROWS_PER_TILE)])

    @pl.when(s == NUM_TILES - 1)
    def _init_tail():
        pltpu.sync_copy(base_hbm.at[c, pl.ds(TAIL_ROW0, TAIL_ROWS)],
                        acc_sh.at[pl.ds(TAIL_ROW0, TAIL_ROWS)])
    plsc.subcore_barrier()

    def scale_chunk(k, rows_ref):
        def group_body(g, carry):
            vv = val_v[k, pl.ds(g * 16, 16)]
            for i in range(16):
                e = g * 16 + i
                v = vv[i]
                for q in range(D // 16):
                    sl = pl.ds(q * 16, 16)
                    rows_ref[e, sl] = rows_ref[e, sl] * v
            return carry

        lax.fori_loop(0, CHUNK // 16, group_body, 0)

    def block_body(b, carry):
        pltpu.sync_copy(src_hbm.at[c, s, pl.ds(b * NB, NB)], src_v)
        pltpu.sync_copy(dst_hbm.at[c, s, pl.ds(b * NB, NB)], dst_v)
        pltpu.sync_copy(val_hbm.at[c, s, pl.ds(b * NB, NB)], val_v)
        pltpu.async_copy(table_hbm.at[src_v.at[0]], rows_a, sem_a)

        def pair_body(q, carry2):
            k0 = 2 * q
            pltpu.make_async_copy(table_hbm.at[src_v.at[k0]], rows_a,
                                  sem_a).wait()
            pltpu.async_copy(table_hbm.at[src_v.at[k0 + 1]], rows_b, sem_b)
            scale_chunk(k0, rows_a)
            pltpu.sync_copy(rows_a, acc_sh.at[dst_v.at[k0]], add=True)
            pltpu.make_async_copy(table_hbm.at[src_v.at[k0 + 1]], rows_b,
                                  sem_b).wait()

            @pl.when(k0 + 2 < NB)
            def _prefetch_next():
                pltpu.async_copy(table_hbm.at[src_v.at[k0 + 2]], rows_a,
                                 sem_a)

            scale_chunk(k0 + 1, rows_b)
            pltpu.sync_copy(rows_b, acc_sh.at[dst_v.at[k0 + 1]], add=True)
            return carry2

        lax.fori_loop(0, NB // 2, pair_body, 0)
        return carry

    lax.fori_loop(0, NUM_BLOCKS, block_body, 0)
    plsc.subcore_barrier()
    pltpu.sync_copy(acc_sh.at[pl.ds(row0, ROWS_PER_TILE)],
                    out_hbm.at[c, pl.ds(row0, ROWS_PER_TILE)])

    @pl.when(s == NUM_TILES - 1)
    def _write_tail():
        pltpu.sync_copy(acc_sh.at[pl.ds(TAIL_ROW0, TAIL_ROWS)],
                        out_hbm.at[c, pl.ds(TAIL_ROW0, TAIL_ROWS)])


_sc_spmm = functools.partial(
    pl.kernel,
    out_type=jax.ShapeDtypeStruct((2, N, D), jnp.float32),
    mesh=plsc.VectorSubcoreMesh(core_axis_name="c", subcore_axis_name="s"),
    scratch_types=[
        pltpu.VMEM((NB, CHUNK), jnp.int32),
        pltpu.VMEM((NB, CHUNK), jnp.int32),
        pltpu.VMEM((NB, CHUNK), jnp.float32),
        pltpu.VMEM((CHUNK, D), jnp.float32),
        pltpu.VMEM((CHUNK, D), jnp.float32),
        pltpu.VMEM_SHARED((N, D), jnp.float32),
        pltpu.SemaphoreType.DMA,
        pltpu.SemaphoreType.DMA,
    ],
)(_sc_body)


def kernel(x_a, x_b, adj_ab_indices, adj_ab_values, adj_ba_indices,
           adj_ba_values, W_rel_ab, w_self_a, bias_a, W_rel_ba, w_self_b,
           bias_b):
    x_cat = jnp.stack([x_a, x_b])
    wself = jnp.stack([w_self_a, w_self_b]) * 0.5
    wrel = jnp.stack([W_rel_ab, W_rel_ba]) * 0.5
    bias = jnp.stack([bias_a, bias_b])

    base, table = _tc_matmuls(x_cat, wself, wrel, bias)
    table_flat = table.reshape(2 * N, D)

    pad = E_PAD - E
    zpad_i = jnp.zeros((pad,), jnp.int32)
    zpad_f = jnp.zeros((pad,), jnp.float32)
    src = jnp.stack([
        jnp.concatenate([adj_ab_indices[1].astype(jnp.int32), zpad_i]),
        jnp.concatenate([adj_ba_indices[1].astype(jnp.int32) + N, zpad_i]),
    ]).reshape(2, NUM_TILES, NUM_CHUNKS, CHUNK)
    dst = jnp.stack([
        jnp.concatenate([adj_ab_indices[0].astype(jnp.int32), zpad_i]),
        jnp.concatenate([adj_ba_indices[0].astype(jnp.int32), zpad_i]),
    ]).reshape(2, NUM_TILES, NUM_CHUNKS, CHUNK)
    val = jnp.stack([
        jnp.concatenate([adj_ab_values, zpad_f]),
        jnp.concatenate([adj_ba_values, zpad_f]),
    ]).reshape(2, NUM_TILES, NUM_CHUNKS, CHUNK)

    out = _sc_spmm(table_flat, base, src, dst, val)
    return (out[0], out[1])


# speedup vs baseline: 1.7618x; 1.0047x over previous
import functools

import jax
import jax.numpy as jnp
from jax import lax
from jax.experimental import pallas as pl
from jax.experimental.pallas import tpu as pltpu
from jax.experimental.pallas import tpu_sc as plsc

N = 10000
E = 320000
D = 128

NUM_TILES = 16
CHUNK = 128
NUM_CHUNKS = 160
EDGES_PER_TILE = CHUNK * NUM_CHUNKS
E_PAD = EDGES_PER_TILE * NUM_TILES
NB = 40
NUM_BLOCKS = NUM_CHUNKS // NB
ROWS_PER_TILE = 624
TAIL_ROW0 = ROWS_PER_TILE * NUM_TILES
TAIL_ROWS = N - TAIL_ROW0

ROW_BLOCK = 2000


def _mm_body(x_self_ref, x_other_ref, wself_ref, wrel_ref, bias_ref,
             base_ref, table_ref):
    xs = x_self_ref[0]
    xo = x_other_ref[0]
    base_ref[0] = (
        jnp.dot(xs, wself_ref[0], preferred_element_type=jnp.float32)
        + bias_ref[0]
    )
    table_ref[0] = jnp.dot(xo, wrel_ref[0], preferred_element_type=jnp.float32)


def _tc_matmuls(x_cat, wself, wrel, bias):
    grid = (2, N // ROW_BLOCK)
    return pl.pallas_call(
        _mm_body,
        grid=grid,
        in_specs=[
            pl.BlockSpec((1, ROW_BLOCK, D), lambda c, r: (c, r, 0)),
            pl.BlockSpec((1, ROW_BLOCK, D), lambda c, r: (1 - c, r, 0)),
            pl.BlockSpec((1, D, D), lambda c, r: (c, 0, 0)),
            pl.BlockSpec((1, D, D), lambda c, r: (c, 0, 0)),
            pl.BlockSpec((1, 1, D), lambda c, r: (c, 0, 0)),
        ],
        out_specs=[
            pl.BlockSpec((1, ROW_BLOCK, D), lambda c, r: (c, r, 0)),
            pl.BlockSpec((1, ROW_BLOCK, D), lambda c, r: (c, r, 0)),
        ],
        out_shape=[
            jax.ShapeDtypeStruct((2, N, D), jnp.float32),
            jax.ShapeDtypeStruct((2, N, D), jnp.float32),
        ],
    )(x_cat, x_cat, wself, wrel, bias)


def _sc_body(table_hbm, base_hbm, src_hbm, dst_hbm, val_hbm, out_hbm,
             src_v, dst_v, val_v, rows_a, rows_b, acc_sh, sem_a, sem_b):
    c = lax.axis_index("c")
    s = lax.axis_index("s")
    row0 = s * ROWS_PER_TILE

    pltpu.sync_copy(base_hbm.at[c, pl.ds(row0, ROWS_PER_TILE)],
                    acc_sh.at[pl.ds(row0, ROWS_PER_TILE)])

    @pl.when(s == NUM_TILES - 1)
    def _init_tail():
        pltpu.sync_copy(base_hbm.at[c, pl.ds(TAIL_ROW0, TAIL_ROWS)],
                        acc_sh.at[pl.ds(TAIL_ROW0, TAIL_ROWS)])
    plsc.subcore_barrier()

    def scale_chunk(k, rows_ref):
        def group_body(g, carry):
            vv = val_v[k, pl.ds(g * 16, 16)]
            for i in range(16):
                e = g * 16 + i
                v = vv[i]
                for q in range(D // 16):
                    sl = pl.ds(q * 16, 16)
                    rows_ref[e, sl] = rows_ref[e, sl] * v
            return carry

        lax.fori_loop(0, CHUNK // 16, group_body, 0)

    def block_body(b, carry):
        pltpu.sync_copy(src_hbm.at[c, s, pl.ds(b * NB, NB)], src_v)
        pltpu.sync_copy(dst_hbm.at[c, s, pl.ds(b * NB, NB)], dst_v)
        pltpu.sync_copy(val_hbm.at[c, s, pl.ds(b * NB, NB)], val_v)
        pltpu.async_copy(table_hbm.at[src_v.at[0]], rows_a, sem_a)

        def pair_body(q, carry2):
            k0 = 2 * q
            pltpu.make_async_copy(table_hbm.at[src_v.at[k0]], rows_a,
                                  sem_a).wait()
            pltpu.async_copy(table_hbm.at[src_v.at[k0 + 1]], rows_b, sem_b)
            scale_chunk(k0, rows_a)
            pltpu.sync_copy(rows_a, acc_sh.at[dst_v.at[k0]], add=True)
            pltpu.make_async_copy(table_hbm.at[src_v.at[k0 + 1]], rows_b,
                                  sem_b).wait()

            @pl.when(k0 + 2 < NB)
            def _prefetch_next():
                pltpu.async_copy(table_hbm.at[src_v.at[k0 + 2]], rows_a,
                                 sem_a)

            scale_chunk(k0 + 1, rows_b)
            pltpu.sync_copy(rows_b, acc_sh.at[dst_v.at[k0 + 1]], add=True)
            return carry2

        lax.fori_loop(0, NB // 2, pair_body, 0)
        return carry

    lax.fori_loop(0, NUM_BLOCKS, block_body, 0)
    plsc.subcore_barrier()
    pltpu.sync_copy(acc_sh.at[pl.ds(row0, ROWS_PER_TILE)],
                    out_hbm.at[c, pl.ds(row0, ROWS_PER_TILE)])

    @pl.when(s == NUM_TILES - 1)
    def _write_tail():
        pltpu.sync_copy(acc_sh.at[pl.ds(TAIL_ROW0, TAIL_ROWS)],
                        out_hbm.at[c, pl.ds(TAIL_ROW0, TAIL_ROWS)])


_sc_spmm = functools.partial(
    pl.kernel,
    out_type=jax.ShapeDtypeStruct((2, N, D), jnp.float32),
    mesh=plsc.VectorSubcoreMesh(core_axis_name="c", subcore_axis_name="s"),
    scratch_types=[
        pltpu.VMEM((NB, CHUNK), jnp.int32),
        pltpu.VMEM((NB, CHUNK), jnp.int32),
        pltpu.VMEM((NB, CHUNK), jnp.float32),
        pltpu.VMEM((CHUNK, D), jnp.float32),
        pltpu.VMEM((CHUNK, D), jnp.float32),
        pltpu.VMEM_SHARED((N, D), jnp.float32),
        pltpu.SemaphoreType.DMA,
        pltpu.SemaphoreType.DMA,
    ],
)(_sc_body)


def kernel(x_a, x_b, adj_ab_indices, adj_ab_values, adj_ba_indices,
           adj_ba_values, W_rel_ab, w_self_a, bias_a, W_rel_ba, w_self_b,
           bias_b):
    x_cat = jnp.stack([x_a, x_b])
    wself = jnp.stack([w_self_a, w_self_b]) * 0.5
    wrel = jnp.stack([W_rel_ab, W_rel_ba]) * 0.5
    bias = jnp.stack([bias_a, bias_b])

    base, table = _tc_matmuls(x_cat, wself, wrel, bias)
    table_flat = table.reshape(2 * N, D)

    pad = E_PAD - E
    zpad_i = jnp.zeros((pad,), jnp.int32)
    zpad_f = jnp.zeros((pad,), jnp.float32)
    src = jnp.stack([
        jnp.concatenate([adj_ab_indices[1].astype(jnp.int32), zpad_i]),
        jnp.concatenate([adj_ba_indices[1].astype(jnp.int32) + N, zpad_i]),
    ]).reshape(2, NUM_TILES, NUM_CHUNKS, CHUNK)
    dst = jnp.stack([
        jnp.concatenate([adj_ab_indices[0].astype(jnp.int32), zpad_i]),
        jnp.concatenate([adj_ba_indices[0].astype(jnp.int32), zpad_i]),
    ]).reshape(2, NUM_TILES, NUM_CHUNKS, CHUNK)
    val = jnp.stack([
        jnp.concatenate([adj_ab_values, zpad_f]),
        jnp.concatenate([adj_ba_values, zpad_f]),
    ]).reshape(2, NUM_TILES, NUM_CHUNKS, CHUNK)

    out = _sc_spmm(table_flat, base, src, dst, val)
    return (out[0], out[1])
